# baseline (speedup 1.0000x reference)
import jax
import jax.numpy as jnp
from jax import lax
from jax.experimental import pallas as pl
from jax.experimental.pallas import tpu as pltpu
from jax.experimental.pallas import tpu_sc as plsc

V_DIM = 1000000
EMB = 32
BATCH = 16384
CTX = 20

NC = 2
NS = 16
NW = NC * NS

LANES_PER_BLK = 8192
SC_COL0 = 5760
SPLIT = SC_COL0 * 128
N_BLKS = SPLIT // LANES_PER_BLK
OUT_TC_ROWS = (N_BLKS + 1) * (LANES_PER_BLK // 4)
ZROW_TC = SPLIT

TCOL_FULL = 7812
K = 4
SC_NG = (TCOL_FULL - SC_COL0) // K
SC_GPW = SC_NG // NW
OUT_SC_3D = SC_NG * 16 + 2 + 2
V_SC = OUT_SC_3D * 1024 // EMB
ZROW_SC = (SC_NG * 16 + 2) * 32


def _tc_transpose_body(i_ref, o_ref):
    q = LANES_PER_BLK // 4

    @pl.when(pl.program_id(0) < N_BLKS)
    def _():
        y = i_ref[...].T
        o_ref[...] = jnp.concatenate(
            [y[0 * q : 1 * q], y[1 * q : 2 * q], y[2 * q : 3 * q], y[3 * q : 4 * q]],
            axis=1,
        )

    @pl.when(pl.program_id(0) == N_BLKS)
    def _():
        o_ref[...] = jnp.zeros((q, 128), jnp.float32)


def _fire_in(tab, buf, sem, g):
    for r in range(4):
        for k in range(K):
            pltpu.async_copy(
                tab.at[pl.ds(8 * r, 8), pl.ds(SPLIT + 512 * g + 128 * k, 128)],
                buf.at[4 * r + k],
                sem,
            )


def _drain_in(tab, buf, sem):
    for t in range(16):
        pltpu.make_async_copy(
            tab.at[pl.ds(0, 8), pl.ds(0, 128)], buf.at[t], sem
        ).wait()


def _extract(in_ref, stag_ref):
    e16 = jnp.arange(16, dtype=jnp.int32)
    s_idx = e16 % 8
    tl = [(e16 // 8) * 4 + k for k in range(K)]
    th = [t + 8 for t in tl]

    def body(lq, carry):
        t_lo = lq // 8
        s_row = lq % 8
        for lr in range(4):
            lv = jnp.full((16,), 0, jnp.int32) + (4 * lq + lr)
            for k in range(K):
                lo = plsc.load_gather(in_ref, [tl[k], s_idx, lv])
                hi = plsc.load_gather(in_ref, [th[k], s_idx, lv])
                stag_ref[4 * k + t_lo, s_row, pl.ds(32 * lr, 16)] = lo
                stag_ref[4 * k + t_lo, s_row, pl.ds(32 * lr + 16, 16)] = hi
        return carry

    lax.fori_loop(0, 32, body, 0)


def _detile_body(tab, tail, out, in0, in1, stag0, stag1, si0, si1, so0, so1):
    wid = lax.axis_index("s") * NC + lax.axis_index("c")
    g_base = wid * SC_GPW

    def fire_out(stag, sem, g):
        pltpu.async_copy(stag, out.at[pl.ds(16 * g, 16)], sem)

    def drain_out(stag, sem):
        pltpu.make_async_copy(out.at[pl.ds(0, 16)], stag, sem).wait()

    _fire_in(tab, in0, si0, g_base)

    @pl.loop(0, SC_GPW // 2)
    def pair(i):
        g0 = g_base + 2 * i
        _fire_in(tab, in1, si1, g0 + 1)
        _drain_in(tab, in0, si0)

        @pl.when(i > 0)
        def _():
            drain_out(stag0, so0)

        _extract(in0, stag0)
        fire_out(stag0, so0, g0)

        @pl.when(i < SC_GPW // 2 - 1)
        def _():
            _fire_in(tab, in0, si0, g0 + 2)

        _drain_in(tab, in1, si1)

        @pl.when(i > 0)
        def _():
            drain_out(stag1, so1)

        _extract(in1, stag1)
        fire_out(stag1, so1, g0 + 1)

    drain_out(stag0, so0)
    drain_out(stag1, so1)

    @pl.when(wid == NW - 1)
    def _():
        _fire_in(tab, in0, si0, SC_NG - 1)
        _drain_in(tab, in0, si0)
        _extract(in0, stag0)
        fire_out(stag0, so0, SC_NG - 1)
        drain_out(stag0, so0)

    @pl.when(wid == NW - 2)
    def _():
        pltpu.sync_copy(tail, stag0.at[pl.ds(0, 2)])
        pltpu.sync_copy(stag0.at[pl.ds(0, 2)], out.at[pl.ds(16 * SC_NG, 2)])

    @pl.when(wid == NW - 3)
    def _():
        z = jnp.zeros((16,), jnp.float32)
        for t in range(2):
            for s in range(8):
                for c in range(8):
                    stag0[t, s, pl.ds(16 * c, 16)] = z
        pltpu.sync_copy(stag0.at[pl.ds(0, 2)], out.at[pl.ds(16 * SC_NG + 2, 2)])


BPW = BATCH // NW
IDX_PER_W = BPW * CTX
IDX_CHUNK = 128
ROWS_PER_STEP = 32
GATHERS_PER_STEP = ROWS_PER_STEP * CTX // IDX_CHUNK
N_STEPS = BPW // ROWS_PER_STEP
IDX_ROWS_PER_W = IDX_PER_W // IDX_CHUNK


def _tree_sum(vs):
    while len(vs) > 1:
        nxt = [vs[k] + vs[k + 1] for k in range(0, len(vs) - 1, 2)]
        if len(vs) % 2:
            nxt.append(vs[-1])
        vs = nxt
    return vs[0]


def _cbow_body(
    x_hbm, tc_hbm, sc_hbm, out_hbm,
    idx_v, idx_w, bufa0, bufa1, bufb0, bufb1, out_v, sem0, sem1,
):
    wid = lax.axis_index("s") * NC + lax.axis_index("c")

    pltpu.sync_copy(x_hbm.at[pl.ds(wid * IDX_ROWS_PER_W, IDX_ROWS_PER_W)], idx_v)

    def remap_row(j, carry):
        for cc in range(IDX_CHUNK // 16):
            v = idx_v[j, pl.ds(16 * cc, 16)]
            w_tc = (v & -LANES_PER_BLK) | (
                (v & (LANES_PER_BLK // 4 - 1)) << 2
            ) | ((v >> 11) & 3)
            m = v < SPLIT
            idx_v[j, pl.ds(16 * cc, 16)] = jnp.where(m, w_tc, ZROW_TC)
            idx_w[j, pl.ds(16 * cc, 16)] = jnp.where(m, ZROW_SC, v - SPLIT)
        return carry

    lax.fori_loop(0, IDX_ROWS_PER_W, remap_row, 0)

    bufas = (bufa0, bufa1)
    bufbs = (bufb0, bufb1)
    sems = (sem0, sem1)

    def fire(step, slot):
        cps = []
        for j in range(GATHERS_PER_STEP):
            row = step * GATHERS_PER_STEP + j
            cps.append(
                pltpu.async_copy(
                    tc_hbm.at[idx_v.at[row]],
                    bufas[slot].at[pl.ds(j * IDX_CHUNK, IDX_CHUNK)],
                    sems[slot],
                )
            )
            cps.append(
                pltpu.async_copy(
                    sc_hbm.at[idx_w.at[row]],
                    bufbs[slot].at[pl.ds(j * IDX_CHUNK, IDX_CHUNK)],
                    sems[slot],
                )
            )
        return cps

    def reduce_step(step, slot):
        bufa = bufas[slot]
        bufb = bufbs[slot]
        inv = jnp.float32(1.0 / CTX)

        def row_body(i, carry):
            base = i * CTX
            lo = [bufa[base + j, 0:16] + bufb[base + j, 0:16] for j in range(CTX)]
            hi = [bufa[base + j, 16:32] + bufb[base + j, 16:32] for j in range(CTX)]
            o = step * ROWS_PER_STEP + i
            out_v[o, 0:16] = _tree_sum(lo) * inv
            out_v[o, 16:32] = _tree_sum(hi) * inv
            return carry

        lax.fori_loop(0, ROWS_PER_STEP, row_body, 0)

    inflight = [fire(0, 0), fire(1, 1)]
    for g in range(N_STEPS):
        slot = g % 2
        for cp in inflight[slot]:
            cp.wait()
        reduce_step(g, slot)
        if g + 2 < N_STEPS:
            inflight[slot] = fire(g + 2, slot)

    pltpu.sync_copy(out_v, out_hbm.at[pl.ds(wid * BPW, BPW)])


@jax.jit
def _cbow(x2d, table):
    mesh = plsc.VectorSubcoreMesh(core_axis_name="c", subcore_axis_name="s")
    tabT = table.T

    lin_sc = pl.kernel(
        _detile_body,
        out_type=jax.ShapeDtypeStruct((OUT_SC_3D, 8, 128), jnp.float32),
        mesh=mesh,
        compiler_params=pltpu.CompilerParams(
            use_tc_tiling_on_sc=True, needs_layout_passes=False
        ),
        scratch_types=[
            pltpu.VMEM((16, 8, 128), jnp.float32),
            pltpu.VMEM((16, 8, 128), jnp.float32),
            pltpu.VMEM((16, 8, 128), jnp.float32),
            pltpu.VMEM((16, 8, 128), jnp.float32),
            pltpu.SemaphoreType.DMA,
            pltpu.SemaphoreType.DMA,
            pltpu.SemaphoreType.DMA,
            pltpu.SemaphoreType.DMA,
        ],
    )(tabT, table[128 * TCOL_FULL :].reshape(2, 8, 128))

    lin_tc = pl.pallas_call(
        _tc_transpose_body,
        grid=(N_BLKS + 1,),
        in_specs=[
            pl.BlockSpec(
                (32, LANES_PER_BLK), lambda i: (0, jnp.minimum(i, N_BLKS - 1))
            )
        ],
        out_specs=pl.BlockSpec((LANES_PER_BLK // 4, 128), lambda i: (i, 0)),
        out_shape=jax.ShapeDtypeStruct((OUT_TC_ROWS, 128), jnp.float32),
    )(tabT)

    tbl_tc = lin_tc.reshape(OUT_TC_ROWS * 4, EMB)
    tbl_sc = lin_sc.reshape(V_SC, EMB)

    return pl.kernel(
        _cbow_body,
        out_type=jax.ShapeDtypeStruct((BATCH, EMB), jnp.float32),
        mesh=mesh,
        compiler_params=pltpu.CompilerParams(use_tc_tiling_on_sc=False),
        scratch_types=[
            pltpu.VMEM((IDX_ROWS_PER_W, IDX_CHUNK), jnp.int32),
            pltpu.VMEM((IDX_ROWS_PER_W, IDX_CHUNK), jnp.int32),
            pltpu.VMEM((ROWS_PER_STEP * CTX, EMB), jnp.float32),
            pltpu.VMEM((ROWS_PER_STEP * CTX, EMB), jnp.float32),
            pltpu.VMEM((ROWS_PER_STEP * CTX, EMB), jnp.float32),
            pltpu.VMEM((ROWS_PER_STEP * CTX, EMB), jnp.float32),
            pltpu.VMEM((BPW, EMB), jnp.float32),
            pltpu.SemaphoreType.DMA,
            pltpu.SemaphoreType.DMA,
        ],
    )(x2d, tbl_tc, tbl_sc)


def kernel(x, table):
    x2d = x.astype(jnp.int32).reshape(BATCH * CTX // IDX_CHUNK, IDX_CHUNK)
    return _cbow(x2d, table)


# speedup vs baseline: 8.1821x; 8.1821x over previous
import jax
import jax.numpy as jnp
from jax import lax
from jax.experimental import pallas as pl
from jax.experimental.pallas import tpu as pltpu
from jax.experimental.pallas import tpu_sc as plsc

V_DIM = 1000000
EMB = 32
BATCH = 16384
CTX = 20

NC = 2
NS = 16
NW = NC * NS

LANES_PER_BLK = 32768
N_BLKS = (V_DIM + LANES_PER_BLK - 1) // LANES_PER_BLK
OUT2_ROWS = N_BLKS * (LANES_PER_BLK // 4)
V_PAD = OUT2_ROWS * 4


def _tc_transpose_body(i_ref, o_ref):
    y = i_ref[...].T
    q = LANES_PER_BLK // 4
    o_ref[...] = jnp.concatenate(
        [y[0 * q : 1 * q], y[1 * q : 2 * q], y[2 * q : 3 * q], y[3 * q : 4 * q]],
        axis=1,
    )


BPW = BATCH // NW
IDX_PER_W = BPW * CTX
IDX_CHUNK = 128
ROWS_PER_STEP = 32
GATHERS_PER_STEP = ROWS_PER_STEP * CTX // IDX_CHUNK
N_STEPS = BPW // ROWS_PER_STEP
IDX_ROWS_PER_W = IDX_PER_W // IDX_CHUNK


def _tree_sum(vs):
    while len(vs) > 1:
        nxt = [vs[k] + vs[k + 1] for k in range(0, len(vs) - 1, 2)]
        if len(vs) % 2:
            nxt.append(vs[-1])
        vs = nxt
    return vs[0]


def _cbow_body(x_hbm, tab_hbm, out_hbm, idx_v, buf0, buf1, out_v, sem0, sem1):
    wid = lax.axis_index("s") * NC + lax.axis_index("c")

    pltpu.sync_copy(x_hbm.at[pl.ds(wid * IDX_ROWS_PER_W, IDX_ROWS_PER_W)], idx_v)

    bufs = (buf0, buf1)
    sems = (sem0, sem1)

    def permute_row(j):
        for cc in range(IDX_CHUNK // 16):
            v = idx_v[j, pl.ds(16 * cc, 16)]
            w = (v & -LANES_PER_BLK) | ((v & (LANES_PER_BLK // 4 - 1)) << 2) | ((v >> 13) & 3)
            idx_v[j, pl.ds(16 * cc, 16)] = w

    def fire(step, slot):
        cps = []
        for j in range(GATHERS_PER_STEP):
            row = step * GATHERS_PER_STEP + j
            permute_row(row)
            cps.append(
                pltpu.async_copy(
                    tab_hbm.at[idx_v.at[row]],
                    bufs[slot].at[pl.ds(j * IDX_CHUNK, IDX_CHUNK)],
                    sems[slot],
                )
            )
        return cps

    def reduce_step(step, slot):
        buf = bufs[slot]
        inv = jnp.float32(1.0 / CTX)

        def row_body(i, carry):
            base = i * CTX
            lo = [buf[base + j, 0:16] for j in range(CTX)]
            hi = [buf[base + j, 16:32] for j in range(CTX)]
            o = step * ROWS_PER_STEP + i
            out_v[o, 0:16] = _tree_sum(lo) * inv
            out_v[o, 16:32] = _tree_sum(hi) * inv
            return carry

        lax.fori_loop(0, ROWS_PER_STEP, row_body, 0)

    inflight = [fire(0, 0), fire(1, 1)]
    for g in range(N_STEPS):
        slot = g % 2
        for cp in inflight[slot]:
            cp.wait()
        reduce_step(g, slot)
        if g + 2 < N_STEPS:
            inflight[slot] = fire(g + 2, slot)

    pltpu.sync_copy(out_v, out_hbm.at[pl.ds(wid * BPW, BPW)])


@jax.jit
def _cbow(x2d, table):
    lin = pl.pallas_call(
        _tc_transpose_body,
        grid=(N_BLKS,),
        in_specs=[pl.BlockSpec((32, LANES_PER_BLK), lambda i: (0, i))],
        out_specs=pl.BlockSpec((LANES_PER_BLK // 4, 128), lambda i: (i, 0)),
        out_shape=jax.ShapeDtypeStruct((OUT2_ROWS, 128), jnp.float32),
    )(table.T)

    tbl = lin.reshape(V_PAD, EMB)

    mesh = plsc.VectorSubcoreMesh(core_axis_name="c", subcore_axis_name="s")
    return pl.kernel(
        _cbow_body,
        out_type=jax.ShapeDtypeStruct((BATCH, EMB), jnp.float32),
        mesh=mesh,
        compiler_params=pltpu.CompilerParams(use_tc_tiling_on_sc=False),
        scratch_types=[
            pltpu.VMEM((IDX_ROWS_PER_W, IDX_CHUNK), jnp.int32),
            pltpu.VMEM((ROWS_PER_STEP * CTX, EMB), jnp.float32),
            pltpu.VMEM((ROWS_PER_STEP * CTX, EMB), jnp.float32),
            pltpu.VMEM((BPW, EMB), jnp.float32),
            pltpu.SemaphoreType.DMA,
            pltpu.SemaphoreType.DMA,
        ],
    )(x2d, tbl)


def kernel(x, table):
    x2d = x.astype(jnp.int32).reshape(BATCH * CTX // IDX_CHUNK, IDX_CHUNK)
    return _cbow(x2d, table)
